# baseline (speedup 1.0000x reference)
import functools

import jax
import jax.numpy as jnp
from jax.experimental import pallas as pl
from jax.experimental.pallas import tpu as pltpu

EPS = 1e-6
_LOG_TINY = -88.0


def _titans_kernel(g_ref, e_ref, q_ref, k_ref, v_ref,
                   w0_ref, gam_ref, bet_ref, o_ref, s_ref, *, nc, hb, d):
    c = pl.program_id(1)

    @pl.when(c == 0)
    def _():
        s_ref[...] = w0_ref[0]

    f32 = jnp.float32
    dn_t = (((1,), (1,)), ((), ()))

    hd = hb * d
    qt = q_ref[0]
    kt = k_ref[0]
    vt = v_ref[0]
    cdim = qt.shape[1]

    rr = jax.lax.broadcasted_iota(jnp.int32, (hb, hd), 1) // d
    hh = jax.lax.broadcasted_iota(jnp.int32, (hb, hd), 0)
    meanmat = (rr == hh).astype(f32) * (1.0 / d)
    selmat = (rr == hh).astype(f32)
    bi = jax.lax.broadcasted_iota(jnp.int32, (hd, hd), 0) // d
    bj = jax.lax.broadcasted_iota(jnp.int32, (hd, hd), 1) // d
    blockdiag = bi == bj

    def seg_mean(x):
        return jnp.dot(meanmat, x, preferred_element_type=f32)

    def seg_bcast(m):
        return jax.lax.dot_general(selmat, m, (((0,), (0,)), ((), ())),
                                   preferred_element_type=f32)

    gam = gam_ref[0]
    bet = bet_ref[0]

    z1 = jnp.dot(w0_ref[0], kt, preferred_element_type=f32)
    mu = seg_mean(z1)
    xc = z1 - seg_bcast(mu)
    var = seg_mean(xc * xc)
    rstd = seg_bcast(jax.lax.rsqrt(var + EPS))
    x_hat = xc * rstd
    y = gam * x_hat + bet + kt
    gxh = (2.0 * (y - vt)) * gam
    z = (gxh - seg_bcast(seg_mean(gxh))
         - x_hat * seg_bcast(seg_mean(gxh * x_hat))) * rstd

    lg = jnp.maximum(jnp.log(g_ref[:, 0, 0, :]), _LOG_TINY)
    ii = jax.lax.broadcasted_iota(jnp.int32, (cdim, cdim), 0)
    jj = jax.lax.broadcasted_iota(jnp.int32, (cdim, cdim), 1)
    tri_low = (ii >= jj).astype(f32)
    causal = ii >= jj
    cl = jax.lax.dot_general(lg, tri_low, dn_t,
                             preferred_element_type=f32)
    cl_cols = jax.lax.dot_general(tri_low, lg, dn_t,
                                  preferred_element_type=f32)
    eta_all = e_ref[:, 0, 0, :]
    sum_lg = jnp.sum(lg, axis=-1, keepdims=True)

    s_old = s_ref[...]
    inter = seg_bcast(jnp.exp(cl)) * jnp.dot(s_old, qt,
                                             preferred_element_type=f32)

    intra = []
    for j in range(hb):
        sl = slice(j * d, (j + 1) * d)
        qk = jax.lax.dot_general(qt[sl], kt[sl],
                                 (((0,), (0,)), ((), ())),
                                 preferred_element_type=f32)
        diff = jnp.where(causal, cl_cols[:, j:j + 1] - cl[j:j + 1, :],
                         _LOG_TINY * 100.0)
        pmat = qk * jnp.exp(diff) * (-eta_all[j:j + 1, :])
        intra.append(jax.lax.dot_general(z[sl], pmat, dn_t,
                                         preferred_element_type=f32))
    zq = inter + jnp.concatenate(intra, axis=0)

    @pl.when(c < nc - 1)
    def _():
        be = jnp.exp(sum_lg - cl) * (-eta_all)
        upd = jax.lax.dot_general(z, seg_bcast(be) * kt, dn_t,
                                  preferred_element_type=f32)
        ptot_row = jax.lax.dot_general(jnp.exp(sum_lg), selmat,
                                       (((0,), (0,)), ((), ())),
                                       preferred_element_type=f32)
        s_ref[...] = ptot_row * s_old + jnp.where(blockdiag, upd, 0.0)

    mu2 = seg_mean(zq)
    xc2 = zq - seg_bcast(mu2)
    var2 = seg_mean(xc2 * xc2)
    zq_hat = xc2 * seg_bcast(jax.lax.rsqrt(var2 + EPS))
    o_ref[0] = gam * zq_hat + bet + qt


@functools.partial(jax.jit, static_argnames=("chunk", "hb"))
def _run(q, k, v, gate, eta, w0, gamma, beta, chunk=128, hb=16):
    b, h, l, d = q.shape
    bh = b * h
    hg = bh // hb
    hd = hb * d
    nc = l // chunk

    qf = q.reshape(hg, hb, l, d).transpose(0, 1, 3, 2).reshape(hg, hd, l)
    kf = k.reshape(hg, hb, l, d).transpose(0, 1, 3, 2).reshape(hg, hd, l)
    vf = v.reshape(hg, hb, l, d).transpose(0, 1, 3, 2).reshape(hg, hd, l)
    g_row = gate.reshape(bh, nc, 1, chunk)
    e_row = eta.reshape(bh, nc, 1, chunk)

    w0t = jnp.broadcast_to(w0[None], (b, h, d, d)).reshape(hg, hb, d, d)
    w0t = w0t.transpose(0, 1, 3, 2)
    eyeb = jnp.eye(hb, dtype=w0.dtype)[None, :, :, None, None]
    w0bd = (w0t[:, :, None] * eyeb).transpose(0, 1, 3, 2, 4).reshape(
        hg, hd, hd)

    gamf = jnp.broadcast_to(
        gamma[None], (b, h, 1, d)).reshape(hg, hd, 1)
    gamf = jnp.broadcast_to(gamf, (hg, hd, chunk))
    betf = jnp.broadcast_to(
        beta[None], (b, h, 1, d)).reshape(hg, hd, 1)
    betf = jnp.broadcast_to(betf, (hg, hd, chunk))

    seq_spec = pl.BlockSpec((1, hd, chunk), lambda i, c: (i, 0, c))
    row_spec = pl.BlockSpec((hb, 1, 1, chunk), lambda i, c: (i, c, 0, 0))
    head_mat = pl.BlockSpec((1, hd, hd), lambda i, c: (i, 0, 0))
    head_vec = pl.BlockSpec((1, hd, chunk), lambda i, c: (i, 0, 0))

    out = pl.pallas_call(
        functools.partial(_titans_kernel, nc=nc, hb=hb, d=d),
        out_shape=jax.ShapeDtypeStruct((hg, hd, l), jnp.float32),
        grid=(hg, nc),
        in_specs=[row_spec, row_spec, seq_spec, seq_spec, seq_spec,
                  head_mat, head_vec, head_vec],
        out_specs=seq_spec,
        scratch_shapes=[pltpu.VMEM((hd, hd), jnp.float32)],
        compiler_params=pltpu.CompilerParams(
            dimension_semantics=("parallel", "arbitrary"),
        ),
        name="titans_memory_gla",
    )(g_row, e_row, qf, kf, vf, w0bd, gamf, betf)
    return out.reshape(hg, hb, d, l).transpose(0, 1, 3, 2).reshape(
        b, h, l, d)


def kernel(q, k, v, gate, eta, W0, gamma, beta):
    return _run(q, k, v, gate, eta, W0, gamma, beta)


# speedup vs baseline: 1.3080x; 1.3080x over previous
import functools

import jax
import jax.numpy as jnp
from jax.experimental import pallas as pl
from jax.experimental.pallas import tpu as pltpu

EPS = 1e-6
_LOG_TINY = -88.0


def _titans_kernel(g_ref, e_ref, q_ref, k_ref, v_ref,
                   w0_ref, gam_ref, bet_ref, o_ref, s_ref, *, nc, hb, d):
    c = pl.program_id(1)

    @pl.when(c == 0)
    def _():
        s_ref[...] = w0_ref[0]

    f32 = jnp.float32
    dn_t = (((1,), (1,)), ((), ()))

    hd = hb * d
    qt = q_ref[0]
    kt = k_ref[0]
    vt = v_ref[0]
    cdim = qt.shape[1]

    rr = jax.lax.broadcasted_iota(jnp.int32, (hb, hd), 1) // d
    hh = jax.lax.broadcasted_iota(jnp.int32, (hb, hd), 0)
    meanmat = (rr == hh).astype(f32) * (1.0 / d)
    selmat = (rr == hh).astype(f32)

    def seg_mean(x):
        return jnp.dot(meanmat, x, preferred_element_type=f32)

    def seg_bcast(m):
        return jax.lax.dot_general(selmat, m, (((0,), (0,)), ((), ())),
                                   preferred_element_type=f32)

    gam = gam_ref[0]
    bet = bet_ref[0]

    pd = 2 * d
    hp = hb // 2
    pmask = (jax.lax.broadcasted_iota(jnp.int32, (pd, pd), 0) // d
             == jax.lax.broadcasted_iota(jnp.int32, (pd, pd), 1) // d)

    z1 = jnp.concatenate(
        [jnp.dot(w0_ref[0, p], kt[p * pd:(p + 1) * pd],
                 preferred_element_type=f32) for p in range(hp)],
        axis=0)
    mu = seg_mean(z1)
    xc = z1 - seg_bcast(mu)
    var = seg_mean(xc * xc)
    rstd = seg_bcast(jax.lax.rsqrt(var + EPS))
    x_hat = xc * rstd
    y = gam * x_hat + bet + kt
    gxh = (2.0 * (y - vt)) * gam
    z = (gxh - seg_bcast(seg_mean(gxh))
         - x_hat * seg_bcast(seg_mean(gxh * x_hat))) * rstd

    lg = jnp.maximum(jnp.log(g_ref[:, 0, 0, :]), _LOG_TINY)
    ii = jax.lax.broadcasted_iota(jnp.int32, (cdim, cdim), 0)
    jj = jax.lax.broadcasted_iota(jnp.int32, (cdim, cdim), 1)
    tri_low = (ii >= jj).astype(f32)
    causal = ii >= jj
    cl = jax.lax.dot_general(lg, tri_low, dn_t,
                             preferred_element_type=f32)
    cl_cols = jax.lax.dot_general(tri_low, lg, dn_t,
                                  preferred_element_type=f32)
    eta_all = e_ref[:, 0, 0, :]
    sum_lg = jnp.sum(lg, axis=-1, keepdims=True)

    inter = seg_bcast(jnp.exp(cl)) * jnp.concatenate(
        [jnp.dot(s_ref[p], qt[p * pd:(p + 1) * pd],
                 preferred_element_type=f32) for p in range(hp)],
        axis=0)

    intra = []
    for j in range(hb):
        sl = slice(j * d, (j + 1) * d)
        qk = jax.lax.dot_general(qt[sl], kt[sl],
                                 (((0,), (0,)), ((), ())),
                                 preferred_element_type=f32)
        diff = jnp.where(causal, cl_cols[:, j:j + 1] - cl[j:j + 1, :],
                         _LOG_TINY * 100.0)
        pmat = qk * jnp.exp(diff) * (-eta_all[j:j + 1, :])
        intra.append(jax.lax.dot_general(z[sl], pmat, dn_t,
                                         preferred_element_type=f32))
    zq = inter + jnp.concatenate(intra, axis=0)

    @pl.when(c < nc - 1)
    def _():
        be = jnp.exp(sum_lg - cl) * (-eta_all)
        bek = seg_bcast(be) * kt
        ptot_row = jax.lax.dot_general(jnp.exp(sum_lg), selmat,
                                       (((0,), (0,)), ((), ())),
                                       preferred_element_type=f32)
        for p in range(hp):
            sl2 = slice(p * pd, (p + 1) * pd)
            upd = jax.lax.dot_general(z[sl2], bek[sl2], dn_t,
                                      preferred_element_type=f32)
            s_ref[p] = (ptot_row[:, sl2] * s_ref[p]
                        + jnp.where(pmask, upd, 0.0))

    mu2 = seg_mean(zq)
    xc2 = zq - seg_bcast(mu2)
    var2 = seg_mean(xc2 * xc2)
    zq_hat = xc2 * seg_bcast(jax.lax.rsqrt(var2 + EPS))
    o_ref[0] = gam * zq_hat + bet + qt


@functools.partial(jax.jit, static_argnames=("chunk", "hb"))
def _run(q, k, v, gate, eta, w0, gamma, beta, chunk=128, hb=8):
    b, h, l, d = q.shape
    bh = b * h
    hg = bh // hb
    hd = hb * d
    nc = l // chunk

    qf = q.reshape(hg, hb, l, d).transpose(0, 1, 3, 2).reshape(hg, hd, l)
    kf = k.reshape(hg, hb, l, d).transpose(0, 1, 3, 2).reshape(hg, hd, l)
    vf = v.reshape(hg, hb, l, d).transpose(0, 1, 3, 2).reshape(hg, hd, l)
    g_row = gate.reshape(bh, nc, 1, chunk)
    e_row = eta.reshape(bh, nc, 1, chunk)

    hp = hb // 2
    pd = 2 * d
    w0t = jnp.broadcast_to(w0[None], (b, h, d, d)).reshape(hg, hp, 2, d, d)
    w0t = w0t.transpose(0, 1, 2, 4, 3)
    eyeb = jnp.eye(2, dtype=w0.dtype)[None, None, :, :, None, None]
    w0bd = (w0t[:, :, :, None] * eyeb).transpose(0, 1, 2, 4, 3, 5).reshape(
        hg, hp, pd, pd)

    gamf = jnp.broadcast_to(
        gamma[None], (b, h, 1, d)).reshape(hg, hd, 1)
    gamf = jnp.broadcast_to(gamf, (hg, hd, chunk))
    betf = jnp.broadcast_to(
        beta[None], (b, h, 1, d)).reshape(hg, hd, 1)
    betf = jnp.broadcast_to(betf, (hg, hd, chunk))

    seq_spec = pl.BlockSpec((1, hd, chunk), lambda i, c: (i, 0, c))
    row_spec = pl.BlockSpec((hb, 1, 1, chunk), lambda i, c: (i, c, 0, 0))
    head_mat = pl.BlockSpec((1, hp, pd, pd), lambda i, c: (i, 0, 0, 0))
    head_vec = pl.BlockSpec((1, hd, chunk), lambda i, c: (i, 0, 0))

    out = pl.pallas_call(
        functools.partial(_titans_kernel, nc=nc, hb=hb, d=d),
        out_shape=jax.ShapeDtypeStruct((hg, hd, l), jnp.float32),
        grid=(hg, nc),
        in_specs=[row_spec, row_spec, seq_spec, seq_spec, seq_spec,
                  head_mat, head_vec, head_vec],
        out_specs=seq_spec,
        scratch_shapes=[pltpu.VMEM((hp, pd, pd), jnp.float32)],
        compiler_params=pltpu.CompilerParams(
            dimension_semantics=("parallel", "arbitrary"),
        ),
        name="titans_memory_gla",
    )(g_row, e_row, qf, kf, vf, w0bd, gamf, betf)
    return out.reshape(hg, hb, d, l).transpose(0, 1, 3, 2).reshape(
        b, h, l, d)


def kernel(q, k, v, gate, eta, W0, gamma, beta):
    return _run(q, k, v, gate, eta, W0, gamma, beta)


# speedup vs baseline: 1.7124x; 1.3092x over previous
import functools

import jax
import jax.numpy as jnp
from jax.experimental import pallas as pl
from jax.experimental.pallas import tpu as pltpu

EPS = 1e-6
_LOG_TINY = -88.0


def _titans_kernel(g_ref, e_ref, q_ref, k_ref, v_ref,
                   w0_ref, gam_ref, bet_ref, o_ref, s_ref, *, nc, hb, d):
    c = pl.program_id(1)

    @pl.when(c == 0)
    def _():
        s_ref[...] = w0_ref[0]

    f32 = jnp.float32
    dn_t = (((1,), (1,)), ((), ()))

    hd = hb * d
    qt = q_ref[0]
    kt = k_ref[0]
    vt = v_ref[0]
    cdim = qt.shape[1]

    rr = jax.lax.broadcasted_iota(jnp.int32, (hb, hd), 1) // d
    hh = jax.lax.broadcasted_iota(jnp.int32, (hb, hd), 0)
    meanmat = (rr == hh).astype(f32) * (1.0 / d)
    selmat = (rr == hh).astype(f32)

    def seg_mean(x):
        return jnp.dot(meanmat, x, preferred_element_type=f32)

    def seg_bcast(m):
        return jax.lax.dot_general(selmat, m, (((0,), (0,)), ((), ())),
                                   preferred_element_type=f32)

    gam = gam_ref[0]
    bet = bet_ref[0]

    pd = 2 * d
    hp = hb // 2
    pmask = (jax.lax.broadcasted_iota(jnp.int32, (pd, pd), 0) // d
             == jax.lax.broadcasted_iota(jnp.int32, (pd, pd), 1) // d)

    z1 = jnp.concatenate(
        [jnp.dot(w0_ref[0, p], kt[p * pd:(p + 1) * pd],
                 preferred_element_type=f32) for p in range(hp)],
        axis=0)
    mu = seg_mean(z1)
    xc = z1 - seg_bcast(mu)
    var = seg_mean(xc * xc)
    rstd = seg_bcast(jax.lax.rsqrt(var + EPS))
    x_hat = xc * rstd
    y = gam * x_hat + bet + kt
    gxh = (2.0 * (y - vt)) * gam
    z = (gxh - seg_bcast(seg_mean(gxh))
         - x_hat * seg_bcast(seg_mean(gxh * x_hat))) * rstd

    lg = jnp.maximum(jnp.log(g_ref[:, 0, 0, :]), _LOG_TINY)
    ii = jax.lax.broadcasted_iota(jnp.int32, (cdim, cdim), 0)
    jj = jax.lax.broadcasted_iota(jnp.int32, (cdim, cdim), 1)
    tri_low = (ii >= jj).astype(f32)
    causal = ii >= jj
    cl = jax.lax.dot_general(lg, tri_low, dn_t,
                             preferred_element_type=f32)
    cl_cols = jax.lax.dot_general(tri_low, lg, dn_t,
                                  preferred_element_type=f32)
    eta_all = e_ref[:, 0, 0, :]
    sum_lg = jnp.sum(lg, axis=-1, keepdims=True)

    inter = seg_bcast(jnp.exp(cl)) * jnp.concatenate(
        [jnp.dot(s_ref[p], qt[p * pd:(p + 1) * pd],
                 preferred_element_type=f32) for p in range(hp)],
        axis=0)

    intra = []
    for j in range(hb):
        sl = slice(j * d, (j + 1) * d)
        qk = jax.lax.dot_general(qt[sl], kt[sl],
                                 (((0,), (0,)), ((), ())),
                                 preferred_element_type=f32)
        diff = jnp.where(causal, cl_cols[:, j:j + 1] - cl[j:j + 1, :],
                         _LOG_TINY * 100.0)
        pmat = qk * jnp.exp(diff) * (-eta_all[j:j + 1, :])
        intra.append(jax.lax.dot_general(z[sl], pmat, dn_t,
                                         preferred_element_type=f32))
    zq = inter + jnp.concatenate(intra, axis=0)

    @pl.when(c < nc - 1)
    def _():
        be = jnp.exp(sum_lg - cl) * (-eta_all)
        bek = seg_bcast(be) * kt
        ptot_row = jax.lax.dot_general(jnp.exp(sum_lg), selmat,
                                       (((0,), (0,)), ((), ())),
                                       preferred_element_type=f32)
        for p in range(hp):
            sl2 = slice(p * pd, (p + 1) * pd)
            upd = jax.lax.dot_general(z[sl2], bek[sl2], dn_t,
                                      preferred_element_type=f32)
            s_ref[p] = (ptot_row[:, sl2] * s_ref[p]
                        + jnp.where(pmask, upd, 0.0))

    mu2 = seg_mean(zq)
    xc2 = zq - seg_bcast(mu2)
    var2 = seg_mean(xc2 * xc2)
    zq_hat = xc2 * seg_bcast(jax.lax.rsqrt(var2 + EPS))
    o_ref[0] = gam * zq_hat + bet + qt


@functools.partial(jax.jit, static_argnames=("chunk", "hb"))
def _run(q, k, v, gate, eta, w0, gamma, beta, chunk=128, hb=16):
    b, h, l, d = q.shape
    bh = b * h
    hg = bh // hb
    hd = hb * d
    nc = l // chunk

    qf = q.reshape(hg, hb, l, d).transpose(0, 1, 3, 2).reshape(hg, hd, l)
    kf = k.reshape(hg, hb, l, d).transpose(0, 1, 3, 2).reshape(hg, hd, l)
    vf = v.reshape(hg, hb, l, d).transpose(0, 1, 3, 2).reshape(hg, hd, l)
    g_row = gate.reshape(bh, nc, 1, chunk)
    e_row = eta.reshape(bh, nc, 1, chunk)

    hp = hb // 2
    pd = 2 * d
    w0t = jnp.broadcast_to(w0[None], (b, h, d, d)).reshape(hg, hp, 2, d, d)
    w0t = w0t.transpose(0, 1, 2, 4, 3)
    eyeb = jnp.eye(2, dtype=w0.dtype)[None, None, :, :, None, None]
    w0bd = (w0t[:, :, :, None] * eyeb).transpose(0, 1, 2, 4, 3, 5).reshape(
        hg, hp, pd, pd)

    gamf = jnp.broadcast_to(
        gamma[None], (b, h, 1, d)).reshape(hg, hd, 1)
    gamf = jnp.broadcast_to(gamf, (hg, hd, chunk))
    betf = jnp.broadcast_to(
        beta[None], (b, h, 1, d)).reshape(hg, hd, 1)
    betf = jnp.broadcast_to(betf, (hg, hd, chunk))

    seq_spec = pl.BlockSpec((1, hd, chunk), lambda i, c: (i, 0, c))
    row_spec = pl.BlockSpec((hb, 1, 1, chunk), lambda i, c: (i, c, 0, 0))
    head_mat = pl.BlockSpec((1, hp, pd, pd), lambda i, c: (i, 0, 0, 0))
    head_vec = pl.BlockSpec((1, hd, chunk), lambda i, c: (i, 0, 0))

    out = pl.pallas_call(
        functools.partial(_titans_kernel, nc=nc, hb=hb, d=d),
        out_shape=jax.ShapeDtypeStruct((hg, hd, l), jnp.float32),
        grid=(hg, nc),
        in_specs=[row_spec, row_spec, seq_spec, seq_spec, seq_spec,
                  head_mat, head_vec, head_vec],
        out_specs=seq_spec,
        scratch_shapes=[pltpu.VMEM((hp, pd, pd), jnp.float32)],
        compiler_params=pltpu.CompilerParams(
            dimension_semantics=("parallel", "arbitrary"),
        ),
        name="titans_memory_gla",
    )(g_row, e_row, qf, kf, vf, w0bd, gamf, betf)
    return out.reshape(hg, hb, d, l).transpose(0, 1, 3, 2).reshape(
        b, h, l, d)


def kernel(q, k, v, gate, eta, W0, gamma, beta):
    return _run(q, k, v, gate, eta, W0, gamma, beta)
